# Initial kernel scaffold; baseline (speedup 1.0000x reference)
import jax
import jax.numpy as jnp
from jax.experimental import pallas as pl


def kernel(x_n0, x_n1, edge_index_0, edge_index_1, params):
    raise NotImplementedError("write your pallas kernel here")


# speedup vs baseline: 2.6709x; 2.6709x over previous
import jax
import jax.numpy as jnp
from jax import lax
from jax.experimental import pallas as pl
from jax.experimental.pallas import tpu as pltpu
from jax.experimental.pallas import tpu_sc as plsc

N = 50000
NP = N + 8
D = 128
E = 600000
EP = 622592
EROWS = EP // 128
NR = 8
QACC = 8192
NACC = NR * QACC
NCNT = 65536
BN = 2000
GRID = N // BN

_MESH = dict(core_axis_name="c", subcore_axis_name="s")


def _agg_pair_body(ta, sa, da, tb, sb, db, outa, outb,
                   sstage, dstage_sb, rows, gstage, dstage, zbuf, acc, sems):
  cid = lax.axis_index("c")
  sid = lax.axis_index("s")

  n_rows = EROWS // 16
  n_sb = n_rows // 16

  for j in range(16):
    for jj in range(8):
      zbuf[j, jj * 16:(jj + 1) * 16] = jnp.zeros((16,), jnp.float32)

  def _zero_acc(k, _):
    pltpu.sync_copy(zbuf, acc.at[pl.ds(sid * (QACC // 16) + k * 16, 16)])
    return 0

  lax.fori_loop(0, QACC // 16 // 16, _zero_acc, 0)
  plsc.subcore_barrier()

  for table, src2d, dst2d, out in ((ta, sa, da, outa), (tb, sb, db, outb)):
    for p in range(NR // 2):
      q = p * 2 + cid
      lo = q * QACC

      def _mask_block(r, k):
        for j in range(8):
          sl = pl.ds(j * 16, 16)
          s16 = sstage[r, sl]
          d16 = dstage_sb[r, sl]
          m = (d16 >= lo) & (d16 < lo + QACC)
          gstage[k, sl] = jnp.where(m, s16, -1)
          dstage[k, sl] = jnp.where(m, d16 - lo, -1)

      def _gather_desc(k):
        return pltpu.make_async_copy(
            table.at[plsc.Indices(gstage.at[k], ignored_value=-1)],
            rows.at[k],
            sems.at[k],
        )

      def _scatter(k):
        pltpu.sync_copy(
            rows.at[k],
            acc.at[plsc.Indices(dstage.at[k], ignored_value=-1)],
            add=True,
        )

      def _sb(g, _):
        base = sid * n_rows + g * 16
        pltpu.sync_copy(src2d.at[pl.ds(base, 16)], sstage)
        pltpu.sync_copy(dst2d.at[pl.ds(base, 16)], dstage_sb)
        for k in range(2):
          _mask_block(k, k)
          _gather_desc(k).start()
        for r in range(14):
          _gather_desc(r % 2).wait()
          _scatter(r % 2)
          _mask_block(r + 2, r % 2)
          _gather_desc(r % 2).start()
        for r in (14, 15):
          _gather_desc(r % 2).wait()
          _scatter(r % 2)
        return 0

      lax.fori_loop(0, n_sb, _sb, 0)
      plsc.subcore_barrier()

      pltpu.sync_copy(
          acc.at[pl.ds(sid * (QACC // 16), QACC // 16)],
          out.at[pl.ds(lo + sid * (QACC // 16), QACC // 16)],
      )
      _zero_again = not (table is tb and p == NR // 2 - 1)
      if _zero_again:
        lax.fori_loop(0, QACC // 16 // 16, _zero_acc, 0)
      plsc.subcore_barrier()


def _sc_agg_pair(ta, ea, tb, eb):
  outs = pl.kernel(
      _agg_pair_body,
      out_type=[jax.ShapeDtypeStruct((NACC, D), jnp.float32),
                jax.ShapeDtypeStruct((NACC, D), jnp.float32)],
      mesh=plsc.VectorSubcoreMesh(**_MESH),
      scratch_types=[
          pltpu.VMEM((16, 128), jnp.int32),
          pltpu.VMEM((16, 128), jnp.int32),
          pltpu.VMEM((2, 128, D), jnp.float32),
          pltpu.VMEM((2, 128), jnp.int32),
          pltpu.VMEM((2, 128), jnp.int32),
          pltpu.VMEM((16, D), jnp.float32),
          pltpu.VMEM_SHARED((QACC, D), jnp.float32),
          pltpu.SemaphoreType.DMA((2,)),
      ],
      compiler_params=pltpu.CompilerParams(needs_layout_passes=False),
  )(ta, ea[0], ea[1], tb, eb[0], eb[1])
  return outs


def _count_pair_body(d2a, d2b, out, dst_v, cnt_v, idxr, accv, sacc):
  cid = lax.axis_index("c")
  sid = lax.axis_index("s")
  wid = cid * 16 + sid

  n_rows = EROWS // 32

  zero16 = jnp.zeros((16,), jnp.float32)
  ones16 = jnp.ones((16,), jnp.float32)
  iota16 = lax.iota(jnp.int32, 16)

  for ch in range(4):
    for j in range(8):
      idxr[ch, j * 16:(j + 1) * 16] = iota16 + (ch * 128 + j * 16)
  for r in range(32):
    for j in range(8):
      accv[r, j * 16:(j + 1) * 16] = zero16

  rows32 = pl.ds(sid * 32, 32)

  for t, dst2d in enumerate((d2a, d2b)):
    pltpu.sync_copy(dst2d.at[pl.ds(wid * n_rows, n_rows)], dst_v)

    def _zero(r, _):
      for j in range(8):
        cnt_v[r, j * 16:(j + 1) * 16] = zero16
      return 0

    lax.fori_loop(0, NCNT // 128, _zero, 0)

    pltpu.sync_copy(accv, sacc.at[rows32])

    def _hist(b, _):
      for j in range(8):
        d16 = dst_v[b, pl.ds(j * 16, 16)]
        plsc.addupdate_scatter(
            cnt_v, [lax.shift_right_logical(d16, 7), d16 & 127], ones16
        )
      return 0

    lax.fori_loop(0, n_rows, _hist, 0)
    plsc.subcore_barrier()

    for ch in range(4):
      pltpu.sync_copy(
          cnt_v.at[pl.ds(ch * 128, 128)],
          sacc.at[plsc.Indices(idxr.at[ch])],
          add=True,
      )
    plsc.subcore_barrier()

    pltpu.sync_copy(sacc.at[rows32], out.at[t, cid, rows32])
    plsc.subcore_barrier()


def _sc_count_pair(d2a, d2b):
  return pl.kernel(
      _count_pair_body,
      out_type=jax.ShapeDtypeStruct((2, 2, NCNT // 128, 128), jnp.float32),
      mesh=plsc.VectorSubcoreMesh(**_MESH),
      scratch_types=[
          pltpu.VMEM((EROWS // 32, 128), jnp.int32),
          pltpu.VMEM((NCNT // 128, 128), jnp.float32),
          pltpu.VMEM((4, 128), jnp.int32),
          pltpu.VMEM((32, 128), jnp.float32),
          pltpu.VMEM_SHARED((NCNT // 128, 128), jnp.float32),
      ],
      compiler_params=pltpu.CompilerParams(needs_layout_passes=False),
  )(d2a, d2b)


def _stats_body(xd, agg, cnt, wd, ws, wul, wur, bd, bs, bu, h_ref, st_ref):
  i = pl.program_id(0)
  wdf = jnp.dot(wul[...], wd[...], preferred_element_type=jnp.float32)
  wsf = jnp.dot(wur[...], ws[...], preferred_element_type=jnp.float32)
  aggm = agg[...] / jnp.maximum(cnt[...], 1.0)
  hd = lax.dot_general(xd[...], wdf, (((1,), (1,)), ((), ())),
                       preferred_element_type=jnp.float32)
  ha = lax.dot_general(aggm, wsf, (((1,), (1,)), ((), ())),
                       preferred_element_type=jnp.float32)
  bf = (bu[...]
        + lax.dot_general(bd[...], wul[...], (((1,), (1,)), ((), ())),
                          preferred_element_type=jnp.float32)
        + lax.dot_general(bs[...], wur[...], (((1,), (1,)), ((), ())),
                          preferred_element_type=jnp.float32))
  h = hd + ha + bf
  h_ref[...] = h
  s = jnp.sum(h, axis=0, keepdims=True)
  ss = jnp.sum(h * h, axis=0, keepdims=True)
  upd = jnp.concatenate([s, ss, jnp.zeros((6, D), jnp.float32)], axis=0)

  @pl.when(i == 0)
  def _():
    st_ref[...] = jnp.zeros_like(st_ref)

  st_ref[...] += upd


def _tc_stats(xd, agg, cnt, wd, ws, wu, bd, bs, bu):
  wul = wu[:, :D]
  wur = wu[:, D:]
  row = lambda v: v.reshape(1, -1)
  return pl.pallas_call(
      _stats_body,
      grid=(GRID,),
      in_specs=[
          pl.BlockSpec((BN, D), lambda i: (i, 0)),
          pl.BlockSpec((BN, D), lambda i: (i, 0)),
          pl.BlockSpec((BN, 1), lambda i: (i, 0)),
          pl.BlockSpec((D, D), lambda i: (0, 0)),
          pl.BlockSpec((D, D), lambda i: (0, 0)),
          pl.BlockSpec((D, D), lambda i: (0, 0)),
          pl.BlockSpec((D, D), lambda i: (0, 0)),
          pl.BlockSpec((1, D), lambda i: (0, 0)),
          pl.BlockSpec((1, D), lambda i: (0, 0)),
          pl.BlockSpec((1, D), lambda i: (0, 0)),
      ],
      out_specs=[
          pl.BlockSpec((BN, D), lambda i: (i, 0)),
          pl.BlockSpec((8, D), lambda i: (0, 0)),
      ],
      out_shape=[
          jax.ShapeDtypeStruct((N, D), jnp.float32),
          jax.ShapeDtypeStruct((8, D), jnp.float32),
      ],
  )(xd, agg, cnt, wd, ws, wul, wur, row(bd), row(bs), row(bu))


def _bn_lrelu(h, st, g, b):
  m = st[0:1, :] / N
  v = st[1:2, :] / N - m * m
  y = g[...] * (h - m) / jnp.sqrt(v + 1.0) + b[...]
  return jnp.where(y >= 0, y, 0.01 * y)


def _apply_mid_body(h_ref, st_ref, g, b, y_ref):
  y_ref[...] = _bn_lrelu(h_ref[...], st_ref[...], g, b)


def _tc_apply_mid(h, st, g, b):
  row = lambda v: v.reshape(1, -1)
  return pl.pallas_call(
      _apply_mid_body,
      grid=(GRID,),
      in_specs=[
          pl.BlockSpec((BN, D), lambda i: (i, 0)),
          pl.BlockSpec((8, D), lambda i: (0, 0)),
          pl.BlockSpec((1, D), lambda i: (0, 0)),
          pl.BlockSpec((1, D), lambda i: (0, 0)),
      ],
      out_specs=pl.BlockSpec((BN, D), lambda i: (i, 0)),
      out_shape=jax.ShapeDtypeStruct((NP, D), jnp.float32),
  )(h, st, row(g), row(b))


def _apply_out_body(h_ref, st_ref, g, b, wp, bp, o_ref):
  y = _bn_lrelu(h_ref[...], st_ref[...], g, b)
  o_ref[...] = lax.dot_general(y, wp[...], (((1,), (1,)), ((), ())),
                               preferred_element_type=jnp.float32) + bp[...]


def _tc_apply_out(h, st, g, b, wp, bp):
  row = lambda v: v.reshape(1, -1)
  L = wp.shape[0]
  return pl.pallas_call(
      _apply_out_body,
      grid=(GRID,),
      in_specs=[
          pl.BlockSpec((BN, D), lambda i: (i, 0)),
          pl.BlockSpec((8, D), lambda i: (0, 0)),
          pl.BlockSpec((1, D), lambda i: (0, 0)),
          pl.BlockSpec((1, D), lambda i: (0, 0)),
          pl.BlockSpec((L, D), lambda i: (0, 0)),
          pl.BlockSpec((1, L), lambda i: (0, 0)),
      ],
      out_specs=pl.BlockSpec((BN, L), lambda i: (i, 0)),
      out_shape=jax.ShapeDtypeStruct((N, L), jnp.float32),
  )(h, st, row(g), row(b), wp, row(bp))


def _pad_edges(idx, spread):
  pad = N + jnp.arange(EP - E, dtype=jnp.int32) % spread
  return jnp.concatenate([idx, pad]).reshape(EROWS, 128)


def _cnt_col(cnt2):
  return (cnt2[0] + cnt2[1]).reshape(NCNT)[:N].reshape(N, 1)


@jax.jit
def kernel(x_n0, x_n1, edge_index_0, edge_index_1, params):
  p = params
  zpad = jnp.zeros((NP - N, D), jnp.float32)
  x0 = jnp.concatenate([x_n0, zpad], axis=0)
  x1 = jnp.concatenate([x_n1, zpad], axis=0)
  src0 = _pad_edges(edge_index_0[0], NP - N)
  dst0 = _pad_edges(edge_index_0[1], NACC - N)
  src1 = _pad_edges(edge_index_1[0], NP - N)
  dst1 = _pad_edges(edge_index_1[1], NACC - N)

  cnts = _sc_count_pair(dst0, dst1)
  cnt_a = _cnt_col(cnts[0])
  cnt_b = _cnt_col(cnts[1])

  agg_a1, agg_b1 = _sc_agg_pair(x0, (src0, dst0), x1, (src1, dst1))

  h1, st1 = _tc_stats(x1, agg_a1, cnt_a, p['W1A_dst'], p['W1A_src'],
                      p['W1A_upd'], p['b1A_dst'], p['b1A_src'], p['b1A_upd'])
  h0, st0 = _tc_stats(x0, agg_b1, cnt_b, p['W1B_dst'], p['W1B_src'],
                      p['W1B_upd'], p['b1B_dst'], p['b1B_src'], p['b1B_upd'])

  y1 = _tc_apply_mid(h1, st1, p['bn1_g_n1'], p['bn1_b_n1'])
  y0 = _tc_apply_mid(h0, st0, p['bn1_g_n0'], p['bn1_b_n0'])

  agg_a2, agg_b2 = _sc_agg_pair(y0, (src0, dst0), y1, (src1, dst1))

  g1, st1b = _tc_stats(y1, agg_a2, cnt_a, p['W2A_dst'], p['W2A_src'],
                       p['W2A_upd'], p['b2A_dst'], p['b2A_src'], p['b2A_upd'])
  g0, st0b = _tc_stats(y0, agg_b2, cnt_b, p['W2B_dst'], p['W2B_src'],
                       p['W2B_upd'], p['b2B_dst'], p['b2B_src'], p['b2B_upd'])

  out0 = _tc_apply_out(g0, st0b, p['bn2_g_n0'], p['bn2_b_n0'],
                       p['Wp_n0'], p['bp_n0'])
  out1 = _tc_apply_out(g1, st1b, p['bn2_g_n1'], p['bn2_b_n1'],
                       p['Wp_n1'], p['bp_n1'])
  return out0, out1


# speedup vs baseline: 2.7486x; 1.0291x over previous
import jax
import jax.numpy as jnp
from jax import lax
from jax.experimental import pallas as pl
from jax.experimental.pallas import tpu as pltpu
from jax.experimental.pallas import tpu_sc as plsc

N = 50000
NP = N + 8
D = 128
E = 600000
EP = 622592
EROWS = EP // 128
NR = 8
QACC = 8192
NACC = NR * QACC
NCNT = 65536
BN = 2000
GRID = N // BN

_MESH = dict(core_axis_name="c", subcore_axis_name="s")


def _agg_body(table, src2d, dst2d, out,
              sstage, dstage_sb, rows, gstage, dstage, zbuf, acc, sems):
  cid = lax.axis_index("c")
  sid = lax.axis_index("s")

  n_rows = EROWS // 16
  n_sb = n_rows // 16

  for j in range(16):
    for jj in range(8):
      zbuf[j, jj * 16:(jj + 1) * 16] = jnp.zeros((16,), jnp.float32)

  def _zero_acc(k, _):
    pltpu.sync_copy(zbuf, acc.at[pl.ds(sid * (QACC // 16) + k * 16, 16)])
    return 0

  lax.fori_loop(0, QACC // 16 // 16, _zero_acc, 0)
  plsc.subcore_barrier()

  if True:
    for p in range(NR // 2):
      q = p * 2 + cid
      lo = q * QACC

      def _mask_block(r, k):
        for j in range(8):
          sl = pl.ds(j * 16, 16)
          s16 = sstage[r, sl]
          d16 = dstage_sb[r, sl]
          m = (d16 >= lo) & (d16 < lo + QACC)
          gstage[k, sl] = jnp.where(m, s16, -1)
          dstage[k, sl] = jnp.where(m, d16 - lo, -1)

      def _gather_desc(k):
        return pltpu.make_async_copy(
            table.at[plsc.Indices(gstage.at[k], ignored_value=-1)],
            rows.at[k],
            sems.at[k],
        )

      def _scatter(k):
        pltpu.sync_copy(
            rows.at[k],
            acc.at[plsc.Indices(dstage.at[k], ignored_value=-1)],
            add=True,
        )

      def _sb(g, _):
        base = sid * n_rows + g * 16
        pltpu.sync_copy(src2d.at[pl.ds(base, 16)], sstage)
        pltpu.sync_copy(dst2d.at[pl.ds(base, 16)], dstage_sb)
        for k in range(2):
          _mask_block(k, k)
          _gather_desc(k).start()
        for r in range(14):
          _gather_desc(r % 2).wait()
          _scatter(r % 2)
          _mask_block(r + 2, r % 2)
          _gather_desc(r % 2).start()
        for r in (14, 15):
          _gather_desc(r % 2).wait()
          _scatter(r % 2)
        return 0

      lax.fori_loop(0, n_sb, _sb, 0)
      plsc.subcore_barrier()

      pltpu.sync_copy(
          acc.at[pl.ds(sid * (QACC // 16), QACC // 16)],
          out.at[pl.ds(lo + sid * (QACC // 16), QACC // 16)],
      )
      if p < NR // 2 - 1:
        lax.fori_loop(0, QACC // 16 // 16, _zero_acc, 0)
      plsc.subcore_barrier()


def _sc_agg(table, src2d, dst2d):
  return pl.kernel(
      _agg_body,
      out_type=jax.ShapeDtypeStruct((NACC, D), jnp.float32),
      mesh=plsc.VectorSubcoreMesh(**_MESH),
      scratch_types=[
          pltpu.VMEM((16, 128), jnp.int32),
          pltpu.VMEM((16, 128), jnp.int32),
          pltpu.VMEM((2, 128, D), jnp.float32),
          pltpu.VMEM((2, 128), jnp.int32),
          pltpu.VMEM((2, 128), jnp.int32),
          pltpu.VMEM((16, D), jnp.float32),
          pltpu.VMEM_SHARED((QACC, D), jnp.float32),
          pltpu.SemaphoreType.DMA((2,)),
      ],
      compiler_params=pltpu.CompilerParams(needs_layout_passes=False),
  )(table, src2d, dst2d)


def _count_body(dst2d, out, dst_v, cnt_v, idxr, accv, sacc):
  cid = lax.axis_index("c")
  sid = lax.axis_index("s")
  wid = cid * 16 + sid

  n_rows = EROWS // 32

  zero16 = jnp.zeros((16,), jnp.float32)
  ones16 = jnp.ones((16,), jnp.float32)
  iota16 = lax.iota(jnp.int32, 16)

  for ch in range(4):
    for j in range(8):
      idxr[ch, j * 16:(j + 1) * 16] = iota16 + (ch * 128 + j * 16)
  for r in range(32):
    for j in range(8):
      accv[r, j * 16:(j + 1) * 16] = zero16

  rows32 = pl.ds(sid * 32, 32)

  if True:
    pltpu.sync_copy(dst2d.at[pl.ds(wid * n_rows, n_rows)], dst_v)

    def _zero(r, _):
      for j in range(8):
        cnt_v[r, j * 16:(j + 1) * 16] = zero16
      return 0

    lax.fori_loop(0, NCNT // 128, _zero, 0)

    pltpu.sync_copy(accv, sacc.at[rows32])

    def _hist(b, _):
      for j in range(8):
        d16 = dst_v[b, pl.ds(j * 16, 16)]
        plsc.addupdate_scatter(
            cnt_v, [lax.shift_right_logical(d16, 7), d16 & 127], ones16
        )
      return 0

    lax.fori_loop(0, n_rows, _hist, 0)
    plsc.subcore_barrier()

    for ch in range(4):
      pltpu.sync_copy(
          cnt_v.at[pl.ds(ch * 128, 128)],
          sacc.at[plsc.Indices(idxr.at[ch])],
          add=True,
      )
    plsc.subcore_barrier()

    pltpu.sync_copy(sacc.at[rows32], out.at[cid, rows32])


def _sc_count(dst2d):
  return pl.kernel(
      _count_body,
      out_type=jax.ShapeDtypeStruct((2, NCNT // 128, 128), jnp.float32),
      mesh=plsc.VectorSubcoreMesh(**_MESH),
      scratch_types=[
          pltpu.VMEM((EROWS // 32, 128), jnp.int32),
          pltpu.VMEM((NCNT // 128, 128), jnp.float32),
          pltpu.VMEM((4, 128), jnp.int32),
          pltpu.VMEM((32, 128), jnp.float32),
          pltpu.VMEM_SHARED((NCNT // 128, 128), jnp.float32),
      ],
      compiler_params=pltpu.CompilerParams(needs_layout_passes=False),
  )(dst2d)


def _stats_body(xd, agg, cnt, wd, ws, wul, wur, bd, bs, bu, h_ref, st_ref):
  i = pl.program_id(0)
  wdf = jnp.dot(wul[...], wd[...], preferred_element_type=jnp.float32)
  wsf = jnp.dot(wur[...], ws[...], preferred_element_type=jnp.float32)
  aggm = agg[...] / jnp.maximum(cnt[...], 1.0)
  hd = lax.dot_general(xd[...], wdf, (((1,), (1,)), ((), ())),
                       preferred_element_type=jnp.float32)
  ha = lax.dot_general(aggm, wsf, (((1,), (1,)), ((), ())),
                       preferred_element_type=jnp.float32)
  bf = (bu[...]
        + lax.dot_general(bd[...], wul[...], (((1,), (1,)), ((), ())),
                          preferred_element_type=jnp.float32)
        + lax.dot_general(bs[...], wur[...], (((1,), (1,)), ((), ())),
                          preferred_element_type=jnp.float32))
  h = hd + ha + bf
  h_ref[...] = h
  s = jnp.sum(h, axis=0, keepdims=True)
  ss = jnp.sum(h * h, axis=0, keepdims=True)
  upd = jnp.concatenate([s, ss, jnp.zeros((6, D), jnp.float32)], axis=0)

  @pl.when(i == 0)
  def _():
    st_ref[...] = jnp.zeros_like(st_ref)

  st_ref[...] += upd


def _tc_stats(xd, agg, cnt, wd, ws, wu, bd, bs, bu):
  wul = wu[:, :D]
  wur = wu[:, D:]
  row = lambda v: v.reshape(1, -1)
  return pl.pallas_call(
      _stats_body,
      grid=(GRID,),
      in_specs=[
          pl.BlockSpec((BN, D), lambda i: (i, 0)),
          pl.BlockSpec((BN, D), lambda i: (i, 0)),
          pl.BlockSpec((BN, 1), lambda i: (i, 0)),
          pl.BlockSpec((D, D), lambda i: (0, 0)),
          pl.BlockSpec((D, D), lambda i: (0, 0)),
          pl.BlockSpec((D, D), lambda i: (0, 0)),
          pl.BlockSpec((D, D), lambda i: (0, 0)),
          pl.BlockSpec((1, D), lambda i: (0, 0)),
          pl.BlockSpec((1, D), lambda i: (0, 0)),
          pl.BlockSpec((1, D), lambda i: (0, 0)),
      ],
      out_specs=[
          pl.BlockSpec((BN, D), lambda i: (i, 0)),
          pl.BlockSpec((8, D), lambda i: (0, 0)),
      ],
      out_shape=[
          jax.ShapeDtypeStruct((N, D), jnp.float32),
          jax.ShapeDtypeStruct((8, D), jnp.float32),
      ],
  )(xd, agg, cnt, wd, ws, wul, wur, row(bd), row(bs), row(bu))


def _bn_lrelu(h, st, g, b):
  m = st[0:1, :] / N
  v = st[1:2, :] / N - m * m
  y = g[...] * (h - m) / jnp.sqrt(v + 1.0) + b[...]
  return jnp.where(y >= 0, y, 0.01 * y)


def _apply_mid_body(h_ref, st_ref, g, b, y_ref):
  y_ref[...] = _bn_lrelu(h_ref[...], st_ref[...], g, b)


def _tc_apply_mid(h, st, g, b):
  row = lambda v: v.reshape(1, -1)
  return pl.pallas_call(
      _apply_mid_body,
      grid=(GRID,),
      in_specs=[
          pl.BlockSpec((BN, D), lambda i: (i, 0)),
          pl.BlockSpec((8, D), lambda i: (0, 0)),
          pl.BlockSpec((1, D), lambda i: (0, 0)),
          pl.BlockSpec((1, D), lambda i: (0, 0)),
      ],
      out_specs=pl.BlockSpec((BN, D), lambda i: (i, 0)),
      out_shape=jax.ShapeDtypeStruct((NP, D), jnp.float32),
  )(h, st, row(g), row(b))


def _apply_out_body(h_ref, st_ref, g, b, wp, bp, o_ref):
  y = _bn_lrelu(h_ref[...], st_ref[...], g, b)
  o_ref[...] = lax.dot_general(y, wp[...], (((1,), (1,)), ((), ())),
                               preferred_element_type=jnp.float32) + bp[...]


def _tc_apply_out(h, st, g, b, wp, bp):
  row = lambda v: v.reshape(1, -1)
  L = wp.shape[0]
  return pl.pallas_call(
      _apply_out_body,
      grid=(GRID,),
      in_specs=[
          pl.BlockSpec((BN, D), lambda i: (i, 0)),
          pl.BlockSpec((8, D), lambda i: (0, 0)),
          pl.BlockSpec((1, D), lambda i: (0, 0)),
          pl.BlockSpec((1, D), lambda i: (0, 0)),
          pl.BlockSpec((L, D), lambda i: (0, 0)),
          pl.BlockSpec((1, L), lambda i: (0, 0)),
      ],
      out_specs=pl.BlockSpec((BN, L), lambda i: (i, 0)),
      out_shape=jax.ShapeDtypeStruct((N, L), jnp.float32),
  )(h, st, row(g), row(b), wp, row(bp))


def _pad_edges(idx, spread):
  pad = N + jnp.arange(EP - E, dtype=jnp.int32) % spread
  return jnp.concatenate([idx, pad]).reshape(EROWS, 128)


def _cnt_col(cnt2):
  return (cnt2[0] + cnt2[1]).reshape(NCNT)[:N].reshape(N, 1)


@jax.jit
def kernel(x_n0, x_n1, edge_index_0, edge_index_1, params):
  p = params
  zpad = jnp.zeros((NP - N, D), jnp.float32)
  x0 = jnp.concatenate([x_n0, zpad], axis=0)
  x1 = jnp.concatenate([x_n1, zpad], axis=0)
  src0 = _pad_edges(edge_index_0[0], NP - N)
  dst0 = _pad_edges(edge_index_0[1], NACC - N)
  src1 = _pad_edges(edge_index_1[0], NP - N)
  dst1 = _pad_edges(edge_index_1[1], NACC - N)

  cnt_a = _cnt_col(_sc_count(dst0))
  cnt_b = _cnt_col(_sc_count(dst1))

  agg_a1 = _sc_agg(x0, src0, dst0)
  agg_b1 = _sc_agg(x1, src1, dst1)

  h1, st1 = _tc_stats(x1, agg_a1, cnt_a, p['W1A_dst'], p['W1A_src'],
                      p['W1A_upd'], p['b1A_dst'], p['b1A_src'], p['b1A_upd'])
  h0, st0 = _tc_stats(x0, agg_b1, cnt_b, p['W1B_dst'], p['W1B_src'],
                      p['W1B_upd'], p['b1B_dst'], p['b1B_src'], p['b1B_upd'])

  y1 = _tc_apply_mid(h1, st1, p['bn1_g_n1'], p['bn1_b_n1'])
  y0 = _tc_apply_mid(h0, st0, p['bn1_g_n0'], p['bn1_b_n0'])

  agg_a2 = _sc_agg(y0, src0, dst0)
  agg_b2 = _sc_agg(y1, src1, dst1)

  g1, st1b = _tc_stats(y1, agg_a2, cnt_a, p['W2A_dst'], p['W2A_src'],
                       p['W2A_upd'], p['b2A_dst'], p['b2A_src'], p['b2A_upd'])
  g0, st0b = _tc_stats(y0, agg_b2, cnt_b, p['W2B_dst'], p['W2B_src'],
                       p['W2B_upd'], p['b2B_dst'], p['b2B_src'], p['b2B_upd'])

  out0 = _tc_apply_out(g0, st0b, p['bn2_g_n0'], p['bn2_b_n0'],
                       p['Wp_n0'], p['bp_n0'])
  out1 = _tc_apply_out(g1, st1b, p['bn2_g_n1'], p['bn2_b_n1'],
                       p['Wp_n1'], p['bp_n1'])
  return out0, out1


# speedup vs baseline: 3.4493x; 1.2549x over previous
import jax
import jax.numpy as jnp
from jax import lax
from jax.experimental import pallas as pl
from jax.experimental.pallas import tpu as pltpu
from jax.experimental.pallas import tpu_sc as plsc

N = 50000
NP = N + 8
D = 128
E = 600000
EP = 622592
EROWS = EP // 128
NR = 8
QACC = 6400
NACC = NR * QACC
NCNT = 65536
BN = 2000
GRID = N // BN

_MESH = dict(core_axis_name="c", subcore_axis_name="s")


def _agg_body(table, src2d, dst2d, out,
              sstage, dstage_sb, rows, gstage, dstage, zbuf, acc,
              sems1, sems2):
  cid = lax.axis_index("c")
  sid = lax.axis_index("s")

  n_rows = EROWS // 16
  n_sb = n_rows // 16

  for j in range(16):
    for jj in range(8):
      zbuf[j, jj * 16:(jj + 1) * 16] = jnp.zeros((16,), jnp.float32)

  def _zero_acc(k, _):
    pltpu.sync_copy(zbuf, acc.at[pl.ds(sid * (QACC // 16) + k * 16, 16)])
    return 0

  lax.fori_loop(0, QACC // 16 // 16, _zero_acc, 0)
  plsc.subcore_barrier()

  for p in range(NR // 2):
    q = p * 2 + cid
    lo = q * QACC

    def _mask_block(r, k):
      for j in range(8):
        sl = pl.ds(j * 16, 16)
        s16 = sstage[r, sl]
        d16 = dstage_sb[r, sl]
        m = (d16 >= lo) & (d16 < lo + QACC)
        gstage[k, sl] = jnp.where(m, s16, -1)
        dstage[k, sl] = jnp.where(m, d16 - lo, -1)

    def _gather_desc(k):
      return pltpu.make_async_copy(
          table.at[plsc.Indices(gstage.at[k], ignored_value=-1)],
          rows.at[k],
          sems1.at[k],
      )

    def _scatter_desc(k):
      return pltpu.make_async_copy(
          rows.at[k],
          acc.at[plsc.Indices(dstage.at[k], ignored_value=-1)],
          sems2.at[k],
      )

    def _sb(g, _):
      base = sid * n_rows + g * 16
      pltpu.sync_copy(src2d.at[pl.ds(base, 16)], sstage)
      pltpu.sync_copy(dst2d.at[pl.ds(base, 16)], dstage_sb)
      for r in range(2):
        _mask_block(r, r)
        _gather_desc(r).start()
      for r in range(14):
        k = r % 4
        _gather_desc(k).wait()
        _scatter_desc(k).start(add=True)
        k2 = (r + 2) % 4
        if r >= 2:
          _scatter_desc(k2).wait()
        _mask_block(r + 2, k2)
        _gather_desc(k2).start()
      for r in (14, 15):
        k = r % 4
        _gather_desc(k).wait()
        _scatter_desc(k).start(add=True)
      for k in range(4):
        _scatter_desc(k).wait()
      return 0

    lax.fori_loop(0, n_sb, _sb, 0)
    plsc.subcore_barrier()

    pltpu.sync_copy(
        acc.at[pl.ds(sid * (QACC // 16), QACC // 16)],
        out.at[pl.ds(lo + sid * (QACC // 16), QACC // 16)],
    )
    if p < NR // 2 - 1:
      lax.fori_loop(0, QACC // 16 // 16, _zero_acc, 0)
    plsc.subcore_barrier()


def _sc_agg(table, src2d, dst2d):
  return pl.kernel(
      _agg_body,
      out_type=jax.ShapeDtypeStruct((NACC, D), jnp.float32),
      mesh=plsc.VectorSubcoreMesh(**_MESH),
      scratch_types=[
          pltpu.VMEM((16, 128), jnp.int32),
          pltpu.VMEM((16, 128), jnp.int32),
          pltpu.VMEM((4, 128, D), jnp.float32),
          pltpu.VMEM((4, 128), jnp.int32),
          pltpu.VMEM((4, 128), jnp.int32),
          pltpu.VMEM((16, D), jnp.float32),
          pltpu.VMEM_SHARED((QACC, D), jnp.float32),
          pltpu.SemaphoreType.DMA((4,)),
          pltpu.SemaphoreType.DMA((4,)),
      ],
      compiler_params=pltpu.CompilerParams(needs_layout_passes=False),
  )(table, src2d, dst2d)


def _count_body(dst2d, out, dst_v, cnt_v, idxr, accv, sacc):
  cid = lax.axis_index("c")
  sid = lax.axis_index("s")
  wid = cid * 16 + sid

  n_rows = EROWS // 32

  zero16 = jnp.zeros((16,), jnp.float32)
  ones16 = jnp.ones((16,), jnp.float32)
  iota16 = lax.iota(jnp.int32, 16)

  for ch in range(4):
    for j in range(8):
      idxr[ch, j * 16:(j + 1) * 16] = iota16 + (ch * 128 + j * 16)
  for r in range(32):
    for j in range(8):
      accv[r, j * 16:(j + 1) * 16] = zero16

  rows32 = pl.ds(sid * 32, 32)

  if True:
    pltpu.sync_copy(dst2d.at[pl.ds(wid * n_rows, n_rows)], dst_v)

    def _zero(r, _):
      for j in range(8):
        cnt_v[r, j * 16:(j + 1) * 16] = zero16
      return 0

    lax.fori_loop(0, NCNT // 128, _zero, 0)

    pltpu.sync_copy(accv, sacc.at[rows32])

    def _hist(b, _):
      for j in range(8):
        d16 = dst_v[b, pl.ds(j * 16, 16)]
        plsc.addupdate_scatter(
            cnt_v, [lax.shift_right_logical(d16, 7), d16 & 127], ones16
        )
      return 0

    lax.fori_loop(0, n_rows, _hist, 0)
    plsc.subcore_barrier()

    for ch in range(4):
      pltpu.sync_copy(
          cnt_v.at[pl.ds(ch * 128, 128)],
          sacc.at[plsc.Indices(idxr.at[ch])],
          add=True,
      )
    plsc.subcore_barrier()

    pltpu.sync_copy(sacc.at[rows32], out.at[cid, rows32])


def _sc_count(dst2d):
  return pl.kernel(
      _count_body,
      out_type=jax.ShapeDtypeStruct((2, NCNT // 128, 128), jnp.float32),
      mesh=plsc.VectorSubcoreMesh(**_MESH),
      scratch_types=[
          pltpu.VMEM((EROWS // 32, 128), jnp.int32),
          pltpu.VMEM((NCNT // 128, 128), jnp.float32),
          pltpu.VMEM((4, 128), jnp.int32),
          pltpu.VMEM((32, 128), jnp.float32),
          pltpu.VMEM_SHARED((NCNT // 128, 128), jnp.float32),
      ],
      compiler_params=pltpu.CompilerParams(needs_layout_passes=False),
  )(dst2d)


def _stats_body(xd, agg, cnt, wd, ws, wul, wur, bd, bs, bu, h_ref, st_ref):
  i = pl.program_id(0)
  wdf = jnp.dot(wul[...], wd[...], preferred_element_type=jnp.float32)
  wsf = jnp.dot(wur[...], ws[...], preferred_element_type=jnp.float32)
  aggm = agg[...] / jnp.maximum(cnt[...], 1.0)
  hd = lax.dot_general(xd[...], wdf, (((1,), (1,)), ((), ())),
                       preferred_element_type=jnp.float32)
  ha = lax.dot_general(aggm, wsf, (((1,), (1,)), ((), ())),
                       preferred_element_type=jnp.float32)
  bf = (bu[...]
        + lax.dot_general(bd[...], wul[...], (((1,), (1,)), ((), ())),
                          preferred_element_type=jnp.float32)
        + lax.dot_general(bs[...], wur[...], (((1,), (1,)), ((), ())),
                          preferred_element_type=jnp.float32))
  h = hd + ha + bf
  h_ref[...] = h
  s = jnp.sum(h, axis=0, keepdims=True)
  ss = jnp.sum(h * h, axis=0, keepdims=True)
  upd = jnp.concatenate([s, ss, jnp.zeros((6, D), jnp.float32)], axis=0)

  @pl.when(i == 0)
  def _():
    st_ref[...] = jnp.zeros_like(st_ref)

  st_ref[...] += upd


def _tc_stats(xd, agg, cnt, wd, ws, wu, bd, bs, bu):
  wul = wu[:, :D]
  wur = wu[:, D:]
  row = lambda v: v.reshape(1, -1)
  return pl.pallas_call(
      _stats_body,
      grid=(GRID,),
      in_specs=[
          pl.BlockSpec((BN, D), lambda i: (i, 0)),
          pl.BlockSpec((BN, D), lambda i: (i, 0)),
          pl.BlockSpec((BN, 1), lambda i: (i, 0)),
          pl.BlockSpec((D, D), lambda i: (0, 0)),
          pl.BlockSpec((D, D), lambda i: (0, 0)),
          pl.BlockSpec((D, D), lambda i: (0, 0)),
          pl.BlockSpec((D, D), lambda i: (0, 0)),
          pl.BlockSpec((1, D), lambda i: (0, 0)),
          pl.BlockSpec((1, D), lambda i: (0, 0)),
          pl.BlockSpec((1, D), lambda i: (0, 0)),
      ],
      out_specs=[
          pl.BlockSpec((BN, D), lambda i: (i, 0)),
          pl.BlockSpec((8, D), lambda i: (0, 0)),
      ],
      out_shape=[
          jax.ShapeDtypeStruct((N, D), jnp.float32),
          jax.ShapeDtypeStruct((8, D), jnp.float32),
      ],
  )(xd, agg, cnt, wd, ws, wul, wur, row(bd), row(bs), row(bu))


def _bn_lrelu(h, st, g, b):
  m = st[0:1, :] / N
  v = st[1:2, :] / N - m * m
  y = g[...] * (h - m) / jnp.sqrt(v + 1.0) + b[...]
  return jnp.where(y >= 0, y, 0.01 * y)


def _apply_mid_body(h_ref, st_ref, g, b, y_ref):
  y_ref[...] = _bn_lrelu(h_ref[...], st_ref[...], g, b)


def _tc_apply_mid(h, st, g, b):
  row = lambda v: v.reshape(1, -1)
  return pl.pallas_call(
      _apply_mid_body,
      grid=(GRID,),
      in_specs=[
          pl.BlockSpec((BN, D), lambda i: (i, 0)),
          pl.BlockSpec((8, D), lambda i: (0, 0)),
          pl.BlockSpec((1, D), lambda i: (0, 0)),
          pl.BlockSpec((1, D), lambda i: (0, 0)),
      ],
      out_specs=pl.BlockSpec((BN, D), lambda i: (i, 0)),
      out_shape=jax.ShapeDtypeStruct((NP, D), jnp.float32),
  )(h, st, row(g), row(b))


def _apply_out_body(h_ref, st_ref, g, b, wp, bp, o_ref):
  y = _bn_lrelu(h_ref[...], st_ref[...], g, b)
  o_ref[...] = lax.dot_general(y, wp[...], (((1,), (1,)), ((), ())),
                               preferred_element_type=jnp.float32) + bp[...]


def _tc_apply_out(h, st, g, b, wp, bp):
  row = lambda v: v.reshape(1, -1)
  L = wp.shape[0]
  return pl.pallas_call(
      _apply_out_body,
      grid=(GRID,),
      in_specs=[
          pl.BlockSpec((BN, D), lambda i: (i, 0)),
          pl.BlockSpec((8, D), lambda i: (0, 0)),
          pl.BlockSpec((1, D), lambda i: (0, 0)),
          pl.BlockSpec((1, D), lambda i: (0, 0)),
          pl.BlockSpec((L, D), lambda i: (0, 0)),
          pl.BlockSpec((1, L), lambda i: (0, 0)),
      ],
      out_specs=pl.BlockSpec((BN, L), lambda i: (i, 0)),
      out_shape=jax.ShapeDtypeStruct((N, L), jnp.float32),
  )(h, st, row(g), row(b), wp, row(bp))


def _pad_edges(idx, spread):
  pad = N + jnp.arange(EP - E, dtype=jnp.int32) % spread
  return jnp.concatenate([idx, pad]).reshape(EROWS, 128)


def _cnt_col(cnt2):
  return (cnt2[0] + cnt2[1]).reshape(NCNT)[:N].reshape(N, 1)


@jax.jit
def kernel(x_n0, x_n1, edge_index_0, edge_index_1, params):
  p = params
  zpad = jnp.zeros((NP - N, D), jnp.float32)
  x0 = jnp.concatenate([x_n0, zpad], axis=0)
  x1 = jnp.concatenate([x_n1, zpad], axis=0)
  src0 = _pad_edges(edge_index_0[0], NP - N)
  dst0 = _pad_edges(edge_index_0[1], NACC - N)
  src1 = _pad_edges(edge_index_1[0], NP - N)
  dst1 = _pad_edges(edge_index_1[1], NACC - N)

  cnt_a = _cnt_col(_sc_count(dst0))
  cnt_b = _cnt_col(_sc_count(dst1))

  agg_a1 = _sc_agg(x0, src0, dst0)
  agg_b1 = _sc_agg(x1, src1, dst1)

  h1, st1 = _tc_stats(x1, agg_a1, cnt_a, p['W1A_dst'], p['W1A_src'],
                      p['W1A_upd'], p['b1A_dst'], p['b1A_src'], p['b1A_upd'])
  h0, st0 = _tc_stats(x0, agg_b1, cnt_b, p['W1B_dst'], p['W1B_src'],
                      p['W1B_upd'], p['b1B_dst'], p['b1B_src'], p['b1B_upd'])

  y1 = _tc_apply_mid(h1, st1, p['bn1_g_n1'], p['bn1_b_n1'])
  y0 = _tc_apply_mid(h0, st0, p['bn1_g_n0'], p['bn1_b_n0'])

  agg_a2 = _sc_agg(y0, src0, dst0)
  agg_b2 = _sc_agg(y1, src1, dst1)

  g1, st1b = _tc_stats(y1, agg_a2, cnt_a, p['W2A_dst'], p['W2A_src'],
                       p['W2A_upd'], p['b2A_dst'], p['b2A_src'], p['b2A_upd'])
  g0, st0b = _tc_stats(y0, agg_b2, cnt_b, p['W2B_dst'], p['W2B_src'],
                       p['W2B_upd'], p['b2B_dst'], p['b2B_src'], p['b2B_upd'])

  out0 = _tc_apply_out(g0, st0b, p['bn2_g_n0'], p['bn2_b_n0'],
                       p['Wp_n0'], p['bp_n0'])
  out1 = _tc_apply_out(g1, st1b, p['bn2_g_n1'], p['bn2_b_n1'],
                       p['Wp_n1'], p['bp_n1'])
  return out0, out1


# speedup vs baseline: 4.8847x; 1.4161x over previous
import jax
import jax.numpy as jnp
from jax import lax
from jax.experimental import pallas as pl
from jax.experimental.pallas import tpu as pltpu
from jax.experimental.pallas import tpu_sc as plsc

N = 50000
NP = N + 8
D = 128
E = 600000
EP = 622592
EROWS = EP // 128
NR = 8
QACC = 6400
NACC = NR * QACC
NCNT = 65536
BN = 2000
GRID = N // BN

_MESH = dict(core_axis_name="c", subcore_axis_name="s")


def _agg_body(table, src2d, dst2d, out,
              sstage, dstage_sb, rows, gbuf, dbuf, zbuf, acc,
              sems1, sems2):
  cid = lax.axis_index("c")
  sid = lax.axis_index("s")

  n_rows = EROWS // 16
  n_sb = n_rows // 16

  for j in range(16):
    for jj in range(8):
      zbuf[j, jj * 16:(jj + 1) * 16] = jnp.zeros((16,), jnp.float32)

  def _zero_acc(k, _):
    pltpu.sync_copy(zbuf, acc.at[pl.ds(sid * (QACC // 16) + k * 16, 16)])
    return 0

  lax.fori_loop(0, QACC // 16 // 16, _zero_acc, 0)
  plsc.subcore_barrier()

  for p in range(NR // 2):
    q = p * 2 + cid
    lo = q * QACC

    def _gather_desc(w, k):
      return pltpu.make_async_copy(
          table.at[plsc.Indices(gbuf.at[pl.ds(w * 128, 128)],
                                ignored_value=-1)],
          rows.at[k],
          sems1.at[k],
      )

    def _scatter_desc(w, k):
      return pltpu.make_async_copy(
          rows.at[k],
          acc.at[plsc.Indices(dbuf.at[pl.ds(w * 128, 128)],
                              ignored_value=-1)],
          sems2.at[k],
      )

    def _sb(g, _):
      base = sid * n_rows + g * 16
      pltpu.sync_copy(src2d.at[pl.ds(base, 16)], sstage)
      pltpu.sync_copy(dst2d.at[pl.ds(base, 16)], dstage_sb)

      goff = jnp.int32(0)
      for r in range(16):
        for j in range(8):
          sl = pl.ds(j * 16, 16)
          s16 = sstage[r, sl]
          d16 = dstage_sb[r, sl]
          m = (d16 >= lo) & (d16 < lo + QACC)
          plsc.store_compressed(gbuf.at[pl.ds(goff, 16)], s16, mask=m)
          plsc.store_compressed(dbuf.at[pl.ds(goff, 16)], d16 - lo, mask=m)
          goff = goff + jnp.sum(m.astype(jnp.int32))
      neg = jnp.full((16,), -1, jnp.int32)
      for j in range(8):
        gbuf[pl.ds(goff + j * 16, 16)] = neg
        dbuf[pl.ds(goff + j * 16, 16)] = neg

      nw = (goff + 127) // 128
      for wg in range(4):
        for k in range(4):
          w = wg * 4 + k

          @pl.when(w < nw)
          def _():
            if wg > 0:
              _scatter_desc(w - 4, k).wait()
            _gather_desc(w, k).start()

        for k in range(4):
          w = wg * 4 + k

          @pl.when(w < nw)
          def _():
            _gather_desc(w, k).wait()
            _scatter_desc(w, k).start(add=True)

      for k in range(4):
        @pl.when(k < nw)
        def _():
          _scatter_desc(k, k).wait()

      return 0

    lax.fori_loop(0, n_sb, _sb, 0)
    plsc.subcore_barrier()

    pltpu.sync_copy(
        acc.at[pl.ds(sid * (QACC // 16), QACC // 16)],
        out.at[pl.ds(lo + sid * (QACC // 16), QACC // 16)],
    )
    if p < NR // 2 - 1:
      lax.fori_loop(0, QACC // 16 // 16, _zero_acc, 0)
    plsc.subcore_barrier()


def _sc_agg(table, src2d, dst2d):
  return pl.kernel(
      _agg_body,
      out_type=jax.ShapeDtypeStruct((NACC, D), jnp.float32),
      mesh=plsc.VectorSubcoreMesh(**_MESH),
      scratch_types=[
          pltpu.VMEM((16, 128), jnp.int32),
          pltpu.VMEM((16, 128), jnp.int32),
          pltpu.VMEM((4, 128, D), jnp.float32),
          pltpu.VMEM((2304,), jnp.int32),
          pltpu.VMEM((2304,), jnp.int32),
          pltpu.VMEM((16, D), jnp.float32),
          pltpu.VMEM_SHARED((QACC, D), jnp.float32),
          pltpu.SemaphoreType.DMA((4,)),
          pltpu.SemaphoreType.DMA((4,)),
      ],
      compiler_params=pltpu.CompilerParams(needs_layout_passes=False),
  )(table, src2d, dst2d)


def _count_body(dst2d, out, dst_v, cnt_v, idxr, accv, sacc):
  cid = lax.axis_index("c")
  sid = lax.axis_index("s")
  wid = cid * 16 + sid

  n_rows = EROWS // 32

  zero16 = jnp.zeros((16,), jnp.float32)
  ones16 = jnp.ones((16,), jnp.float32)
  iota16 = lax.iota(jnp.int32, 16)

  for ch in range(4):
    for j in range(8):
      idxr[ch, j * 16:(j + 1) * 16] = iota16 + (ch * 128 + j * 16)
  for r in range(32):
    for j in range(8):
      accv[r, j * 16:(j + 1) * 16] = zero16

  rows32 = pl.ds(sid * 32, 32)

  if True:
    pltpu.sync_copy(dst2d.at[pl.ds(wid * n_rows, n_rows)], dst_v)

    def _zero(r, _):
      for j in range(8):
        cnt_v[r, j * 16:(j + 1) * 16] = zero16
      return 0

    lax.fori_loop(0, NCNT // 128, _zero, 0)

    pltpu.sync_copy(accv, sacc.at[rows32])

    def _hist(b, _):
      for j in range(8):
        d16 = dst_v[b, pl.ds(j * 16, 16)]
        plsc.addupdate_scatter(
            cnt_v, [lax.shift_right_logical(d16, 7), d16 & 127], ones16
        )
      return 0

    lax.fori_loop(0, n_rows, _hist, 0)
    plsc.subcore_barrier()

    for ch in range(4):
      pltpu.sync_copy(
          cnt_v.at[pl.ds(ch * 128, 128)],
          sacc.at[plsc.Indices(idxr.at[ch])],
          add=True,
      )
    plsc.subcore_barrier()

    pltpu.sync_copy(sacc.at[rows32], out.at[cid, rows32])


def _sc_count(dst2d):
  return pl.kernel(
      _count_body,
      out_type=jax.ShapeDtypeStruct((2, NCNT // 128, 128), jnp.float32),
      mesh=plsc.VectorSubcoreMesh(**_MESH),
      scratch_types=[
          pltpu.VMEM((EROWS // 32, 128), jnp.int32),
          pltpu.VMEM((NCNT // 128, 128), jnp.float32),
          pltpu.VMEM((4, 128), jnp.int32),
          pltpu.VMEM((32, 128), jnp.float32),
          pltpu.VMEM_SHARED((NCNT // 128, 128), jnp.float32),
      ],
      compiler_params=pltpu.CompilerParams(needs_layout_passes=False),
  )(dst2d)


def _stats_body(xd, agg, cnt, wd, ws, wul, wur, bd, bs, bu, h_ref, st_ref):
  i = pl.program_id(0)
  wdf = jnp.dot(wul[...], wd[...], preferred_element_type=jnp.float32)
  wsf = jnp.dot(wur[...], ws[...], preferred_element_type=jnp.float32)
  aggm = agg[...] / jnp.maximum(cnt[...], 1.0)
  hd = lax.dot_general(xd[...], wdf, (((1,), (1,)), ((), ())),
                       preferred_element_type=jnp.float32)
  ha = lax.dot_general(aggm, wsf, (((1,), (1,)), ((), ())),
                       preferred_element_type=jnp.float32)
  bf = (bu[...]
        + lax.dot_general(bd[...], wul[...], (((1,), (1,)), ((), ())),
                          preferred_element_type=jnp.float32)
        + lax.dot_general(bs[...], wur[...], (((1,), (1,)), ((), ())),
                          preferred_element_type=jnp.float32))
  h = hd + ha + bf
  h_ref[...] = h
  s = jnp.sum(h, axis=0, keepdims=True)
  ss = jnp.sum(h * h, axis=0, keepdims=True)
  upd = jnp.concatenate([s, ss, jnp.zeros((6, D), jnp.float32)], axis=0)

  @pl.when(i == 0)
  def _():
    st_ref[...] = jnp.zeros_like(st_ref)

  st_ref[...] += upd


def _tc_stats(xd, agg, cnt, wd, ws, wu, bd, bs, bu):
  wul = wu[:, :D]
  wur = wu[:, D:]
  row = lambda v: v.reshape(1, -1)
  return pl.pallas_call(
      _stats_body,
      grid=(GRID,),
      in_specs=[
          pl.BlockSpec((BN, D), lambda i: (i, 0)),
          pl.BlockSpec((BN, D), lambda i: (i, 0)),
          pl.BlockSpec((BN, 1), lambda i: (i, 0)),
          pl.BlockSpec((D, D), lambda i: (0, 0)),
          pl.BlockSpec((D, D), lambda i: (0, 0)),
          pl.BlockSpec((D, D), lambda i: (0, 0)),
          pl.BlockSpec((D, D), lambda i: (0, 0)),
          pl.BlockSpec((1, D), lambda i: (0, 0)),
          pl.BlockSpec((1, D), lambda i: (0, 0)),
          pl.BlockSpec((1, D), lambda i: (0, 0)),
      ],
      out_specs=[
          pl.BlockSpec((BN, D), lambda i: (i, 0)),
          pl.BlockSpec((8, D), lambda i: (0, 0)),
      ],
      out_shape=[
          jax.ShapeDtypeStruct((N, D), jnp.float32),
          jax.ShapeDtypeStruct((8, D), jnp.float32),
      ],
  )(xd, agg, cnt, wd, ws, wul, wur, row(bd), row(bs), row(bu))


def _bn_lrelu(h, st, g, b):
  m = st[0:1, :] / N
  v = st[1:2, :] / N - m * m
  y = g[...] * (h - m) / jnp.sqrt(v + 1.0) + b[...]
  return jnp.where(y >= 0, y, 0.01 * y)


def _apply_mid_body(h_ref, st_ref, g, b, y_ref):
  y_ref[...] = _bn_lrelu(h_ref[...], st_ref[...], g, b)


def _tc_apply_mid(h, st, g, b):
  row = lambda v: v.reshape(1, -1)
  return pl.pallas_call(
      _apply_mid_body,
      grid=(GRID,),
      in_specs=[
          pl.BlockSpec((BN, D), lambda i: (i, 0)),
          pl.BlockSpec((8, D), lambda i: (0, 0)),
          pl.BlockSpec((1, D), lambda i: (0, 0)),
          pl.BlockSpec((1, D), lambda i: (0, 0)),
      ],
      out_specs=pl.BlockSpec((BN, D), lambda i: (i, 0)),
      out_shape=jax.ShapeDtypeStruct((NP, D), jnp.float32),
  )(h, st, row(g), row(b))


def _apply_out_body(h_ref, st_ref, g, b, wp, bp, o_ref):
  y = _bn_lrelu(h_ref[...], st_ref[...], g, b)
  o_ref[...] = lax.dot_general(y, wp[...], (((1,), (1,)), ((), ())),
                               preferred_element_type=jnp.float32) + bp[...]


def _tc_apply_out(h, st, g, b, wp, bp):
  row = lambda v: v.reshape(1, -1)
  L = wp.shape[0]
  return pl.pallas_call(
      _apply_out_body,
      grid=(GRID,),
      in_specs=[
          pl.BlockSpec((BN, D), lambda i: (i, 0)),
          pl.BlockSpec((8, D), lambda i: (0, 0)),
          pl.BlockSpec((1, D), lambda i: (0, 0)),
          pl.BlockSpec((1, D), lambda i: (0, 0)),
          pl.BlockSpec((L, D), lambda i: (0, 0)),
          pl.BlockSpec((1, L), lambda i: (0, 0)),
      ],
      out_specs=pl.BlockSpec((BN, L), lambda i: (i, 0)),
      out_shape=jax.ShapeDtypeStruct((N, L), jnp.float32),
  )(h, st, row(g), row(b), wp, row(bp))


def _pad_edges(idx, spread):
  pad = N + jnp.arange(EP - E, dtype=jnp.int32) % spread
  return jnp.concatenate([idx, pad]).reshape(EROWS, 128)


def _cnt_col(cnt2):
  return (cnt2[0] + cnt2[1]).reshape(NCNT)[:N].reshape(N, 1)


@jax.jit
def kernel(x_n0, x_n1, edge_index_0, edge_index_1, params):
  p = params
  zpad = jnp.zeros((NP - N, D), jnp.float32)
  x0 = jnp.concatenate([x_n0, zpad], axis=0)
  x1 = jnp.concatenate([x_n1, zpad], axis=0)
  src0 = _pad_edges(edge_index_0[0], NP - N)
  dst0 = _pad_edges(edge_index_0[1], NACC - N)
  src1 = _pad_edges(edge_index_1[0], NP - N)
  dst1 = _pad_edges(edge_index_1[1], NACC - N)

  cnt_a = _cnt_col(_sc_count(dst0))
  cnt_b = _cnt_col(_sc_count(dst1))

  agg_a1 = _sc_agg(x0, src0, dst0)
  agg_b1 = _sc_agg(x1, src1, dst1)

  h1, st1 = _tc_stats(x1, agg_a1, cnt_a, p['W1A_dst'], p['W1A_src'],
                      p['W1A_upd'], p['b1A_dst'], p['b1A_src'], p['b1A_upd'])
  h0, st0 = _tc_stats(x0, agg_b1, cnt_b, p['W1B_dst'], p['W1B_src'],
                      p['W1B_upd'], p['b1B_dst'], p['b1B_src'], p['b1B_upd'])

  y1 = _tc_apply_mid(h1, st1, p['bn1_g_n1'], p['bn1_b_n1'])
  y0 = _tc_apply_mid(h0, st0, p['bn1_g_n0'], p['bn1_b_n0'])

  agg_a2 = _sc_agg(y0, src0, dst0)
  agg_b2 = _sc_agg(y1, src1, dst1)

  g1, st1b = _tc_stats(y1, agg_a2, cnt_a, p['W2A_dst'], p['W2A_src'],
                       p['W2A_upd'], p['b2A_dst'], p['b2A_src'], p['b2A_upd'])
  g0, st0b = _tc_stats(y0, agg_b2, cnt_b, p['W2B_dst'], p['W2B_src'],
                       p['W2B_upd'], p['b2B_dst'], p['b2B_src'], p['b2B_upd'])

  out0 = _tc_apply_out(g0, st0b, p['bn2_g_n0'], p['bn2_b_n0'],
                       p['Wp_n0'], p['bp_n0'])
  out1 = _tc_apply_out(g1, st1b, p['bn2_g_n1'], p['bn2_b_n1'],
                       p['Wp_n1'], p['bp_n1'])
  return out0, out1


# speedup vs baseline: 6.1525x; 1.2595x over previous
import jax
import jax.numpy as jnp
from jax import lax
from jax.experimental import pallas as pl
from jax.experimental.pallas import tpu as pltpu
from jax.experimental.pallas import tpu_sc as plsc

N = 50000
NP = N + 8
D = 128
E = 600000
EP = 622592
EROWS = EP // 128
NR = 8
QACC = 6400
NACC = NR * QACC
NCNT = 65536
BN = 2000
GRID = N // BN

_MESH = dict(core_axis_name="c", subcore_axis_name="s")


def _agg_body(table, src2d, dst2d, out,
              sstage, dstage_sb, rows, gbuf, dbuf, zbuf, acc,
              sems1, sems2, sems3):
  cid = lax.axis_index("c")
  sid = lax.axis_index("s")

  n_rows = EROWS // 16
  n_sb = n_rows // 16

  for j in range(16):
    for jj in range(8):
      zbuf[j, jj * 16:(jj + 1) * 16] = jnp.zeros((16,), jnp.float32)

  def _zero_acc(k, _):
    pltpu.sync_copy(zbuf, acc.at[pl.ds(sid * (QACC // 16) + k * 16, 16)])
    return 0

  lax.fori_loop(0, QACC // 16 // 16, _zero_acc, 0)
  plsc.subcore_barrier()

  for p in range(NR // 2):
    q = p * 2 + cid
    lo = q * QACC

    def _gather_desc(w, k):
      return pltpu.make_async_copy(
          table.at[plsc.Indices(gbuf.at[pl.ds(w * 128, 128)],
                                ignored_value=-1)],
          rows.at[k],
          sems1.at[k],
      )

    def _scatter_desc(w, k):
      return pltpu.make_async_copy(
          rows.at[k],
          acc.at[plsc.Indices(dbuf.at[pl.ds(w * 128, 128)],
                              ignored_value=-1)],
          sems2.at[k],
      )

    def _idx_descs(g, par):
      base = sid * n_rows + g * 16
      return (
          pltpu.make_async_copy(src2d.at[pl.ds(base, 16)], sstage.at[par],
                                sems3.at[2 * par]),
          pltpu.make_async_copy(dst2d.at[pl.ds(base, 16)], dstage_sb.at[par],
                                sems3.at[2 * par + 1]),
      )

    for d in _idx_descs(0, 0):
      d.start()

    def _sb(g, _):
      par = g % 2
      for d in _idx_descs(g, par):
        d.wait()

      @pl.when(g + 1 < n_sb)
      def _():
        for d in _idx_descs(g + 1, (g + 1) % 2):
          d.start()

      goff = jnp.int32(0)
      for r in range(16):
        svecs = [sstage[par, r, pl.ds(j * 16, 16)] for j in range(8)]
        dvecs = [dstage_sb[par, r, pl.ds(j * 16, 16)] for j in range(8)]
        masks = [(d >= lo) & (d < lo + QACC) for d in dvecs]
        pcs = [jnp.sum(m.astype(jnp.int32)) for m in masks]
        offs = [goff]
        for j in range(7):
          offs.append(offs[-1] + pcs[j])
        goff = offs[-1] + pcs[7]
        for j in range(8):
          plsc.store_compressed(gbuf.at[pl.ds(offs[j], 16)], svecs[j],
                                mask=masks[j])
          plsc.store_compressed(dbuf.at[pl.ds(offs[j], 16)],
                                dvecs[j] - lo, mask=masks[j])
      neg = jnp.full((16,), -1, jnp.int32)
      for j in range(8):
        gbuf[pl.ds(goff + j * 16, 16)] = neg
        dbuf[pl.ds(goff + j * 16, 16)] = neg

      nw = (goff + 127) // 128
      for wg in range(6):
        for k in range(3):
          w = wg * 3 + k

          @pl.when(w < nw)
          def _():
            if wg > 0:
              _scatter_desc(w - 3, k).wait()
            _gather_desc(w, k).start()

        for k in range(3):
          w = wg * 3 + k

          @pl.when(w < nw)
          def _():
            _gather_desc(w, k).wait()
            _scatter_desc(w, k).start(add=True)

      for k in range(3):
        @pl.when(k < nw)
        def _():
          _scatter_desc(k, k).wait()

      return 0

    lax.fori_loop(0, n_sb, _sb, 0)
    plsc.subcore_barrier()

    pltpu.sync_copy(
        acc.at[pl.ds(sid * (QACC // 16), QACC // 16)],
        out.at[pl.ds(lo + sid * (QACC // 16), QACC // 16)],
    )
    if p < NR // 2 - 1:
      lax.fori_loop(0, QACC // 16 // 16, _zero_acc, 0)
    plsc.subcore_barrier()


def _sc_agg(table, src2d, dst2d):
  return pl.kernel(
      _agg_body,
      out_type=jax.ShapeDtypeStruct((NACC, D), jnp.float32),
      mesh=plsc.VectorSubcoreMesh(**_MESH),
      scratch_types=[
          pltpu.VMEM((2, 16, 128), jnp.int32),
          pltpu.VMEM((2, 16, 128), jnp.int32),
          pltpu.VMEM((3, 128, D), jnp.float32),
          pltpu.VMEM((2304,), jnp.int32),
          pltpu.VMEM((2304,), jnp.int32),
          pltpu.VMEM((16, D), jnp.float32),
          pltpu.VMEM_SHARED((QACC, D), jnp.float32),
          pltpu.SemaphoreType.DMA((3,)),
          pltpu.SemaphoreType.DMA((3,)),
          pltpu.SemaphoreType.DMA((4,)),
      ],
      compiler_params=pltpu.CompilerParams(needs_layout_passes=False),
  )(table, src2d, dst2d)


def _count_body(dst2d, out, dst_v, cnt_v, idxr, accv, sacc):
  cid = lax.axis_index("c")
  sid = lax.axis_index("s")
  wid = cid * 16 + sid

  n_rows = EROWS // 32

  zero16 = jnp.zeros((16,), jnp.float32)
  ones16 = jnp.ones((16,), jnp.float32)
  iota16 = lax.iota(jnp.int32, 16)

  for ch in range(4):
    for j in range(8):
      idxr[ch, j * 16:(j + 1) * 16] = iota16 + (ch * 128 + j * 16)
  for r in range(32):
    for j in range(8):
      accv[r, j * 16:(j + 1) * 16] = zero16

  rows32 = pl.ds(sid * 32, 32)

  if True:
    pltpu.sync_copy(dst2d.at[pl.ds(wid * n_rows, n_rows)], dst_v)

    def _zero(r, _):
      for j in range(8):
        cnt_v[r, j * 16:(j + 1) * 16] = zero16
      return 0

    lax.fori_loop(0, NCNT // 128, _zero, 0)

    pltpu.sync_copy(accv, sacc.at[rows32])

    def _hist(b, _):
      for j in range(8):
        d16 = dst_v[b, pl.ds(j * 16, 16)]
        plsc.addupdate_scatter(
            cnt_v, [lax.shift_right_logical(d16, 7), d16 & 127], ones16
        )
      return 0

    lax.fori_loop(0, n_rows, _hist, 0)
    plsc.subcore_barrier()

    for ch in range(4):
      pltpu.sync_copy(
          cnt_v.at[pl.ds(ch * 128, 128)],
          sacc.at[plsc.Indices(idxr.at[ch])],
          add=True,
      )
    plsc.subcore_barrier()

    pltpu.sync_copy(sacc.at[rows32], out.at[cid, rows32])


def _sc_count(dst2d):
  return pl.kernel(
      _count_body,
      out_type=jax.ShapeDtypeStruct((2, NCNT // 128, 128), jnp.float32),
      mesh=plsc.VectorSubcoreMesh(**_MESH),
      scratch_types=[
          pltpu.VMEM((EROWS // 32, 128), jnp.int32),
          pltpu.VMEM((NCNT // 128, 128), jnp.float32),
          pltpu.VMEM((4, 128), jnp.int32),
          pltpu.VMEM((32, 128), jnp.float32),
          pltpu.VMEM_SHARED((NCNT // 128, 128), jnp.float32),
      ],
      compiler_params=pltpu.CompilerParams(needs_layout_passes=False),
  )(dst2d)


def _stats_body(xd, agg, cnt, wd, ws, wul, wur, bd, bs, bu, h_ref, st_ref):
  i = pl.program_id(0)
  wdf = jnp.dot(wul[...], wd[...], preferred_element_type=jnp.float32)
  wsf = jnp.dot(wur[...], ws[...], preferred_element_type=jnp.float32)
  aggm = agg[...] / jnp.maximum(cnt[...], 1.0)
  hd = lax.dot_general(xd[...], wdf, (((1,), (1,)), ((), ())),
                       preferred_element_type=jnp.float32)
  ha = lax.dot_general(aggm, wsf, (((1,), (1,)), ((), ())),
                       preferred_element_type=jnp.float32)
  bf = (bu[...]
        + lax.dot_general(bd[...], wul[...], (((1,), (1,)), ((), ())),
                          preferred_element_type=jnp.float32)
        + lax.dot_general(bs[...], wur[...], (((1,), (1,)), ((), ())),
                          preferred_element_type=jnp.float32))
  h = hd + ha + bf
  h_ref[...] = h
  s = jnp.sum(h, axis=0, keepdims=True)
  ss = jnp.sum(h * h, axis=0, keepdims=True)
  upd = jnp.concatenate([s, ss, jnp.zeros((6, D), jnp.float32)], axis=0)

  @pl.when(i == 0)
  def _():
    st_ref[...] = jnp.zeros_like(st_ref)

  st_ref[...] += upd


def _tc_stats(xd, agg, cnt, wd, ws, wu, bd, bs, bu):
  wul = wu[:, :D]
  wur = wu[:, D:]
  row = lambda v: v.reshape(1, -1)
  return pl.pallas_call(
      _stats_body,
      grid=(GRID,),
      in_specs=[
          pl.BlockSpec((BN, D), lambda i: (i, 0)),
          pl.BlockSpec((BN, D), lambda i: (i, 0)),
          pl.BlockSpec((BN, 1), lambda i: (i, 0)),
          pl.BlockSpec((D, D), lambda i: (0, 0)),
          pl.BlockSpec((D, D), lambda i: (0, 0)),
          pl.BlockSpec((D, D), lambda i: (0, 0)),
          pl.BlockSpec((D, D), lambda i: (0, 0)),
          pl.BlockSpec((1, D), lambda i: (0, 0)),
          pl.BlockSpec((1, D), lambda i: (0, 0)),
          pl.BlockSpec((1, D), lambda i: (0, 0)),
      ],
      out_specs=[
          pl.BlockSpec((BN, D), lambda i: (i, 0)),
          pl.BlockSpec((8, D), lambda i: (0, 0)),
      ],
      out_shape=[
          jax.ShapeDtypeStruct((N, D), jnp.float32),
          jax.ShapeDtypeStruct((8, D), jnp.float32),
      ],
  )(xd, agg, cnt, wd, ws, wul, wur, row(bd), row(bs), row(bu))


def _bn_lrelu(h, st, g, b):
  m = st[0:1, :] / N
  v = st[1:2, :] / N - m * m
  y = g[...] * (h - m) / jnp.sqrt(v + 1.0) + b[...]
  return jnp.where(y >= 0, y, 0.01 * y)


def _apply_mid_body(h_ref, st_ref, g, b, y_ref):
  y_ref[...] = _bn_lrelu(h_ref[...], st_ref[...], g, b)


def _tc_apply_mid(h, st, g, b):
  row = lambda v: v.reshape(1, -1)
  return pl.pallas_call(
      _apply_mid_body,
      grid=(GRID,),
      in_specs=[
          pl.BlockSpec((BN, D), lambda i: (i, 0)),
          pl.BlockSpec((8, D), lambda i: (0, 0)),
          pl.BlockSpec((1, D), lambda i: (0, 0)),
          pl.BlockSpec((1, D), lambda i: (0, 0)),
      ],
      out_specs=pl.BlockSpec((BN, D), lambda i: (i, 0)),
      out_shape=jax.ShapeDtypeStruct((NP, D), jnp.float32),
  )(h, st, row(g), row(b))


def _apply_out_body(h_ref, st_ref, g, b, wp, bp, o_ref):
  y = _bn_lrelu(h_ref[...], st_ref[...], g, b)
  o_ref[...] = lax.dot_general(y, wp[...], (((1,), (1,)), ((), ())),
                               preferred_element_type=jnp.float32) + bp[...]


def _tc_apply_out(h, st, g, b, wp, bp):
  row = lambda v: v.reshape(1, -1)
  L = wp.shape[0]
  return pl.pallas_call(
      _apply_out_body,
      grid=(GRID,),
      in_specs=[
          pl.BlockSpec((BN, D), lambda i: (i, 0)),
          pl.BlockSpec((8, D), lambda i: (0, 0)),
          pl.BlockSpec((1, D), lambda i: (0, 0)),
          pl.BlockSpec((1, D), lambda i: (0, 0)),
          pl.BlockSpec((L, D), lambda i: (0, 0)),
          pl.BlockSpec((1, L), lambda i: (0, 0)),
      ],
      out_specs=pl.BlockSpec((BN, L), lambda i: (i, 0)),
      out_shape=jax.ShapeDtypeStruct((N, L), jnp.float32),
  )(h, st, row(g), row(b), wp, row(bp))


def _pad_edges(idx, spread):
  pad = N + jnp.arange(EP - E, dtype=jnp.int32) % spread
  return jnp.concatenate([idx, pad]).reshape(EROWS, 128)


def _cnt_col(cnt2):
  return (cnt2[0] + cnt2[1]).reshape(NCNT)[:N].reshape(N, 1)


@jax.jit
def kernel(x_n0, x_n1, edge_index_0, edge_index_1, params):
  p = params
  zpad = jnp.zeros((NP - N, D), jnp.float32)
  x0 = jnp.concatenate([x_n0, zpad], axis=0)
  x1 = jnp.concatenate([x_n1, zpad], axis=0)
  src0 = _pad_edges(edge_index_0[0], NP - N)
  dst0 = _pad_edges(edge_index_0[1], NACC - N)
  src1 = _pad_edges(edge_index_1[0], NP - N)
  dst1 = _pad_edges(edge_index_1[1], NACC - N)

  cnt_a = _cnt_col(_sc_count(dst0))
  cnt_b = _cnt_col(_sc_count(dst1))

  agg_a1 = _sc_agg(x0, src0, dst0)
  agg_b1 = _sc_agg(x1, src1, dst1)

  h1, st1 = _tc_stats(x1, agg_a1, cnt_a, p['W1A_dst'], p['W1A_src'],
                      p['W1A_upd'], p['b1A_dst'], p['b1A_src'], p['b1A_upd'])
  h0, st0 = _tc_stats(x0, agg_b1, cnt_b, p['W1B_dst'], p['W1B_src'],
                      p['W1B_upd'], p['b1B_dst'], p['b1B_src'], p['b1B_upd'])

  y1 = _tc_apply_mid(h1, st1, p['bn1_g_n1'], p['bn1_b_n1'])
  y0 = _tc_apply_mid(h0, st0, p['bn1_g_n0'], p['bn1_b_n0'])

  agg_a2 = _sc_agg(y0, src0, dst0)
  agg_b2 = _sc_agg(y1, src1, dst1)

  g1, st1b = _tc_stats(y1, agg_a2, cnt_a, p['W2A_dst'], p['W2A_src'],
                       p['W2A_upd'], p['b2A_dst'], p['b2A_src'], p['b2A_upd'])
  g0, st0b = _tc_stats(y0, agg_b2, cnt_b, p['W2B_dst'], p['W2B_src'],
                       p['W2B_upd'], p['b2B_dst'], p['b2B_src'], p['b2B_upd'])

  out0 = _tc_apply_out(g0, st0b, p['bn2_g_n0'], p['bn2_b_n0'],
                       p['Wp_n0'], p['bp_n0'])
  out1 = _tc_apply_out(g1, st1b, p['bn2_g_n1'], p['bn2_b_n1'],
                       p['Wp_n1'], p['bp_n1'])
  return out0, out1
